# baseline (speedup 1.0000x reference)
import functools

import jax
import jax.numpy as jnp
from jax import lax
from jax.experimental import pallas as pl
from jax.experimental.pallas import tpu as pltpu
from jax.experimental.pallas import tpu_sc as plsc

N_TOKENS = 8192
D_MODEL = 4096
NUM_GROUPS = 16
BLOCK_N = 1024
_HALF = BLOCK_N // 2

_SCORE_STRIDE = 128

_LANES = 16
_NUM_WORKERS = 32
_TOK_PER_WORKER = N_TOKENS // _NUM_WORKERS


def _scores_kernel(xa_ref, xb_ref, w_ref, b_ref, o_ref):
    w = w_ref[...]
    dn = (((1,), (1,)), ((), ()))
    sa = lax.dot_general(xa_ref[...], w, dn,
                         preferred_element_type=jnp.float32)
    sb = lax.dot_general(xb_ref[...], w, dn,
                         preferred_element_type=jnp.float32)
    bias = b_ref[...]
    o_ref[:_HALF, :NUM_GROUPS] = sa + bias
    o_ref[_HALF:, :NUM_GROUPS] = sb + bias


def _tc_scores(x, w2, b2):
    grid = N_TOKENS // BLOCK_N
    return pl.pallas_call(
        _scores_kernel,
        grid=(grid,),
        in_specs=[
            pl.BlockSpec((_HALF, D_MODEL), lambda i: (2 * i, 0)),
            pl.BlockSpec((_HALF, D_MODEL), lambda i: (2 * i + 1, 0)),
            pl.BlockSpec((NUM_GROUPS, D_MODEL), lambda i: (0, 0)),
            pl.BlockSpec((1, NUM_GROUPS), lambda i: (0, 0)),
        ],
        out_specs=pl.BlockSpec((BLOCK_N, _SCORE_STRIDE), lambda i: (i, 0)),
        out_shape=jax.ShapeDtypeStruct((N_TOKENS, _SCORE_STRIDE), jnp.float32),
    )(x, x, w2, b2)


def _sc_select(scores, group):
    mesh = plsc.VectorSubcoreMesh(core_axis_name="c", subcore_axis_name="s")

    @functools.partial(
        pl.kernel, mesh=mesh,
        out_type=jax.ShapeDtypeStruct((N_TOKENS,), jnp.float32),
        scratch_types=[
            pltpu.VMEM((_TOK_PER_WORKER,), jnp.int32),
            pltpu.VMEM((_TOK_PER_WORKER,), jnp.int32),
            pltpu.VMEM((_TOK_PER_WORKER,), jnp.float32),
            pltpu.SemaphoreType.DMA,
        ],
    )
    def sel(scores_hbm, group_hbm, out_hbm, g_v, idx_v, y_v, sem):
        wid = lax.axis_index("s") * 2 + lax.axis_index("c")
        base = wid * _TOK_PER_WORKER
        pltpu.sync_copy(group_hbm.at[pl.ds(base, _TOK_PER_WORKER)], g_v)
        lane = lax.broadcasted_iota(jnp.int32, (_LANES,), 0)
        for i in range(_TOK_PER_WORKER // _LANES):
            cols = g_v[pl.ds(i * _LANES, _LANES)]
            idx_v[pl.ds(i * _LANES, _LANES)] = (
                (lane + (base + i * _LANES)) * _SCORE_STRIDE + cols)
        pltpu.async_copy(scores_hbm.at[idx_v], y_v, sem).wait()
        pltpu.sync_copy(y_v, out_hbm.at[pl.ds(base, _TOK_PER_WORKER)])

    return sel(scores.reshape(-1), group)


def kernel(x, group, W, b):
    g1 = group.astype(jnp.int32)
    w2 = W.reshape(NUM_GROUPS, D_MODEL)
    b2 = b.reshape(1, NUM_GROUPS)
    scores = _tc_scores(x, w2, b2)
    y = _sc_select(scores, g1)
    return y.reshape(N_TOKENS, 1)


# speedup vs baseline: 1.0330x; 1.0330x over previous
import functools

import jax
import jax.numpy as jnp
from jax import lax
from jax.experimental import pallas as pl
from jax.experimental.pallas import tpu as pltpu
from jax.experimental.pallas import tpu_sc as plsc

N_TOKENS = 8192
D_MODEL = 4096
NUM_GROUPS = 16
BLOCK_N = 512
_HALF = BLOCK_N // 2

_SCORE_STRIDE = 128

_LANES = 16
_NUM_WORKERS = 32
_TOK_PER_WORKER = N_TOKENS // _NUM_WORKERS


def _scores_kernel(xa_ref, xb_ref, w_ref, b_ref, o_ref):
    w = w_ref[...]
    dn = (((1,), (1,)), ((), ()))
    sa = lax.dot_general(xa_ref[...], w, dn,
                         preferred_element_type=jnp.float32)
    sb = lax.dot_general(xb_ref[...], w, dn,
                         preferred_element_type=jnp.float32)
    bias = b_ref[...]
    o_ref[:_HALF, :NUM_GROUPS] = sa + bias
    o_ref[_HALF:, :NUM_GROUPS] = sb + bias


def _tc_scores(x, w2, b2):
    grid = N_TOKENS // BLOCK_N
    return pl.pallas_call(
        _scores_kernel,
        grid=(grid,),
        in_specs=[
            pl.BlockSpec((_HALF, D_MODEL), lambda i: (2 * i, 0)),
            pl.BlockSpec((_HALF, D_MODEL), lambda i: (2 * i + 1, 0)),
            pl.BlockSpec((NUM_GROUPS, D_MODEL), lambda i: (0, 0)),
            pl.BlockSpec((1, NUM_GROUPS), lambda i: (0, 0)),
        ],
        out_specs=pl.BlockSpec((BLOCK_N, _SCORE_STRIDE), lambda i: (i, 0)),
        out_shape=jax.ShapeDtypeStruct((N_TOKENS, _SCORE_STRIDE), jnp.float32),
    )(x, x, w2, b2)


def _sc_select(scores, group):
    mesh = plsc.VectorSubcoreMesh(core_axis_name="c", subcore_axis_name="s")

    @functools.partial(
        pl.kernel, mesh=mesh,
        out_type=jax.ShapeDtypeStruct((N_TOKENS,), jnp.float32),
        scratch_types=[
            pltpu.VMEM((_TOK_PER_WORKER,), jnp.int32),
            pltpu.VMEM((_TOK_PER_WORKER,), jnp.int32),
            pltpu.VMEM((_TOK_PER_WORKER,), jnp.float32),
            pltpu.SemaphoreType.DMA,
        ],
    )
    def sel(scores_hbm, group_hbm, out_hbm, g_v, idx_v, y_v, sem):
        wid = lax.axis_index("s") * 2 + lax.axis_index("c")
        base = wid * _TOK_PER_WORKER
        pltpu.sync_copy(group_hbm.at[pl.ds(base, _TOK_PER_WORKER)], g_v)
        lane = lax.broadcasted_iota(jnp.int32, (_LANES,), 0)
        for i in range(_TOK_PER_WORKER // _LANES):
            cols = g_v[pl.ds(i * _LANES, _LANES)]
            idx_v[pl.ds(i * _LANES, _LANES)] = (
                (lane + (base + i * _LANES)) * _SCORE_STRIDE + cols)
        pltpu.async_copy(scores_hbm.at[idx_v], y_v, sem).wait()
        pltpu.sync_copy(y_v, out_hbm.at[pl.ds(base, _TOK_PER_WORKER)])

    return sel(scores.reshape(-1), group)


def kernel(x, group, W, b):
    g1 = group.astype(jnp.int32)
    w2 = W.reshape(NUM_GROUPS, D_MODEL)
    b2 = b.reshape(1, NUM_GROUPS)
    scores = _tc_scores(x, w2, b2)
    y = _sc_select(scores, g1)
    return y.reshape(N_TOKENS, 1)
